# baseline (speedup 1.0000x reference)
import functools

import jax
import jax.numpy as jnp
from jax import lax
from jax.experimental import pallas as pl
from jax.experimental.pallas import tpu as pltpu

_NEG = -1e30


def _gat_body(nb, x_ref, seg_ref, wl_ref, bl_ref, tok_ref, wr_ref, br_ref,
              att_ref, bias_ref, out_ref, num_ref, den_ref, m_ref):
    i = pl.program_id(0)

    @pl.when(i == 0)
    def _init():
        num_ref[...] = jnp.zeros(num_ref.shape, jnp.float32)
        den_ref[...] = jnp.zeros(den_ref.shape, jnp.float32)
        m_ref[0, 0] = _NEG

    nseg = den_ref.shape[0]
    r = x_ref.shape[0]
    seg = seg_ref[...]
    x = x_ref[...]
    xl = jnp.dot(x, wl_ref[...], preferred_element_type=jnp.float32) + bl_ref[...]
    valid = seg < nseg
    xl_z = jnp.where(valid, xl, 0.0)
    xr = jnp.dot(tok_ref[...], wr_ref[...], preferred_element_type=jnp.float32) + br_ref[...]
    ms = xl + xr
    s = jnp.where(ms > 0, ms, 0.2 * ms)
    logit = jnp.sum(s * att_ref[...], axis=1, keepdims=True)
    logit = jnp.where(valid, logit, _NEG)

    c = jnp.max(logit)
    m_old = m_ref[0, 0]
    m_new = jnp.maximum(m_old, c)

    @pl.when(c > m_old)
    def _rescale():
        sc = jnp.exp(m_old - m_new)
        den_ref[...] = den_ref[...] * sc
        num_ref[...] = num_ref[...] * sc

    e = jnp.exp(logit - m_new)
    onehot = seg == lax.broadcasted_iota(jnp.int32, (r, nseg), 1)
    me = jnp.where(onehot, e, 0.0)
    ones_r = jnp.ones((r, 1), jnp.float32)
    den_ref[...] = den_ref[...] + lax.dot_general(
        me, ones_r, (((0,), (0,)), ((), ())), preferred_element_type=jnp.float32)
    num_ref[...] = num_ref[...] + lax.dot_general(
        me, xl_z, (((0,), (0,)), ((), ())), preferred_element_type=jnp.float32)
    m_ref[0, 0] = m_new

    @pl.when(i == nb - 1)
    def _finish():
        out_ref[...] = num_ref[...] / den_ref[...] + bias_ref[...]


def _gat(x, seg, tok, wl, bl, wr, br, att, bias, nseg, block_rows=1024):
    n, nc = x.shape
    gc = wl.shape[1]
    nb = -(-n // block_rows)
    pad = nb * block_rows - n
    segp = jnp.concatenate(
        [seg.astype(jnp.int32), jnp.full((pad,), nseg, jnp.int32)]).reshape(-1, 1)
    out_t = pl.pallas_call(
        functools.partial(_gat_body, nb),
        grid=(nb,),
        in_specs=[
            pl.BlockSpec((block_rows, nc), lambda i: (i, 0)),
            pl.BlockSpec((block_rows, 1), lambda i: (i, 0)),
            pl.BlockSpec((nc, gc), lambda i: (0, 0)),
            pl.BlockSpec((1, gc), lambda i: (0, 0)),
            pl.BlockSpec((1, gc), lambda i: (0, 0)),
            pl.BlockSpec((gc, gc), lambda i: (0, 0)),
            pl.BlockSpec((1, gc), lambda i: (0, 0)),
            pl.BlockSpec((1, gc), lambda i: (0, 0)),
            pl.BlockSpec((1, gc), lambda i: (0, 0)),
        ],
        out_specs=pl.BlockSpec((nseg, gc), lambda i: (0, 0)),
        out_shape=jax.ShapeDtypeStruct((nseg, gc), jnp.float32),
        scratch_shapes=[
            pltpu.VMEM((nseg, gc), jnp.float32),
            pltpu.VMEM((nseg, 1), jnp.float32),
            pltpu.SMEM((1, 1), jnp.float32),
        ],
        compiler_params=pltpu.CompilerParams(dimension_semantics=("arbitrary",)),
    )(x, segp, wl, bl.reshape(1, gc), tok.reshape(1, gc), wr,
      br.reshape(1, gc), att.reshape(1, gc), bias.reshape(1, gc))
    return out_t


def _head_body(ga_ref, go_ref, gm_ref, gv_ref,
               w11, b11, w12, b12, w1p, b1p, g1, bb1,
               w21, b21, w22, b22, g2, bb2,
               w31, b31, w32, b32, w3p, b3p, out_ref):
    def mm(a, w):
        return jnp.dot(a, w[...], preferred_element_type=jnp.float32)

    def bn_tanh(h, g, b):
        mu = jnp.mean(h, axis=0, keepdims=True)
        var = jnp.mean((h - mu) ** 2, axis=0, keepdims=True)
        return jnp.tanh((h - mu) / jnp.sqrt(var + 1e-5) * g[...] + b[...])

    h0 = jnp.concatenate(
        [ga_ref[...], go_ref[...], gm_ref[...], gv_ref[...]], axis=1)
    h = mm(jnp.tanh(mm(h0, w11) + b11[...]), w12) + b12[...]
    h = h + mm(h0, w1p) + b1p[...]
    h = bn_tanh(h, g1, bb1)
    h = h + mm(jnp.tanh(mm(h, w21) + b21[...]), w22) + b22[...]
    h = bn_tanh(h, g2, bb2)
    h3 = mm(jnp.tanh(mm(h, w31) + b31[...]), w32) + b32[...]
    out_ref[...] = h3 + mm(h, w3p) + b3p[...]


def kernel(x_operation, x_machine, x_AGV, global_attr,
           batch_operation, batch_machine, batch_AGV, params):
    p = params
    b = global_attr.shape[0]
    globs = {}
    for t, x, seg in (("operation", x_operation, batch_operation),
                      ("machine", x_machine, batch_machine),
                      ("AGV", x_AGV, batch_AGV)):
        out_t = _gat(x, seg, p["tok_" + t], p["Wl_" + t], p["bl_" + t],
                     p["Wr_" + t], p["br_" + t], p["att_" + t],
                     p["bias_" + t], b)
        globs[t] = out_t

    def r1(name):
        return p[name].reshape(1, -1)

    ggc = p["rl3_W1"].shape[1]
    gf = pl.pallas_call(
        _head_body,
        out_shape=jax.ShapeDtypeStruct((b, ggc), jnp.float32),
    )(global_attr, globs["operation"], globs["machine"], globs["AGV"],
      p["rl1_W1"], r1("rl1_b1"), p["rl1_W2"], r1("rl1_b2"),
      p["rl1_Wp"], r1("rl1_bp"), r1("bn1_g"), r1("bn1_b"),
      p["rl2_W1"], r1("rl2_b1"), p["rl2_W2"], r1("rl2_b2"),
      r1("bn2_g"), r1("bn2_b"),
      p["rl3_W1"], r1("rl3_b1"), p["rl3_W2"], r1("rl3_b2"),
      p["rl3_Wp"], r1("rl3_bp"))
    return (globs["operation"], globs["machine"], globs["AGV"], gf)


# speedup vs baseline: 1.3411x; 1.3411x over previous
import functools

import jax
import jax.numpy as jnp
from jax import lax
from jax.experimental import pallas as pl
from jax.experimental.pallas import tpu as pltpu

_NEG = -1e30


def _gat_body(nb, x_ref, seg_ref, wl_ref, bl_ref, tok_ref, wr_ref, br_ref,
              att_ref, bias_ref, out_ref, num_ref, den_ref, m_ref):
    i = pl.program_id(0)

    @pl.when(i == 0)
    def _init():
        num_ref[...] = jnp.zeros(num_ref.shape, jnp.float32)
        den_ref[...] = jnp.zeros(den_ref.shape, jnp.float32)
        m_ref[0, 0] = _NEG

    nseg = den_ref.shape[1]
    r = x_ref.shape[0]
    seg = seg_ref[...]
    x = x_ref[...]
    xl = jnp.dot(x, wl_ref[...], preferred_element_type=jnp.float32) + bl_ref[...]
    valid = seg < nseg
    xl_z = jnp.where(valid, xl, 0.0)
    xr = jnp.dot(tok_ref[...], wr_ref[...], preferred_element_type=jnp.float32) + br_ref[...]
    ms = xl + xr
    s = jnp.where(ms > 0, ms, 0.2 * ms)
    logit = jnp.sum(s * att_ref[...], axis=1, keepdims=True)
    logit = jnp.where(valid, logit, _NEG)

    c = jnp.max(logit)
    m_old = m_ref[0, 0]
    m_new = jnp.maximum(m_old, c)

    @pl.when(c > m_old)
    def _rescale():
        sc = jnp.exp(m_old - m_new)
        den_ref[...] = den_ref[...] * sc
        num_ref[...] = num_ref[...] * sc

    e = jnp.exp(logit - m_new)
    onehot = seg == lax.broadcasted_iota(jnp.int32, (r, nseg), 1)
    me = jnp.where(onehot, e, 0.0)
    ones_r = jnp.ones((1, r), jnp.float32)
    den_ref[...] = den_ref[...] + jnp.dot(
        ones_r, me, preferred_element_type=jnp.float32)
    num_ref[...] = num_ref[...] + lax.dot_general(
        xl_z, me, (((0,), (0,)), ((), ())), preferred_element_type=jnp.float32)
    m_ref[0, 0] = m_new

    @pl.when(i == nb - 1)
    def _finish():
        out_ref[...] = num_ref[...] / den_ref[...] + bias_ref[...]


def _gat(x, seg, tok, wl, bl, wr, br, att, bias, nseg, block_rows=1024):
    n, nc = x.shape
    gc = wl.shape[1]
    nb = -(-n // block_rows)
    pad = nb * block_rows - n
    segp = jnp.concatenate(
        [seg.astype(jnp.int32), jnp.full((pad,), nseg, jnp.int32)]).reshape(-1, 1)
    out_t = pl.pallas_call(
        functools.partial(_gat_body, nb),
        grid=(nb,),
        in_specs=[
            pl.BlockSpec((block_rows, nc), lambda i: (i, 0)),
            pl.BlockSpec((block_rows, 1), lambda i: (i, 0)),
            pl.BlockSpec((nc, gc), lambda i: (0, 0)),
            pl.BlockSpec((1, gc), lambda i: (0, 0)),
            pl.BlockSpec((1, gc), lambda i: (0, 0)),
            pl.BlockSpec((gc, gc), lambda i: (0, 0)),
            pl.BlockSpec((1, gc), lambda i: (0, 0)),
            pl.BlockSpec((1, gc), lambda i: (0, 0)),
            pl.BlockSpec((gc, 1), lambda i: (0, 0)),
        ],
        out_specs=pl.BlockSpec((gc, nseg), lambda i: (0, 0)),
        out_shape=jax.ShapeDtypeStruct((gc, nseg), jnp.float32),
        scratch_shapes=[
            pltpu.VMEM((gc, nseg), jnp.float32),
            pltpu.VMEM((1, nseg), jnp.float32),
            pltpu.SMEM((1, 1), jnp.float32),
        ],
        compiler_params=pltpu.CompilerParams(dimension_semantics=("arbitrary",)),
    )(x, segp, wl, bl.reshape(1, gc), tok.reshape(1, gc), wr,
      br.reshape(1, gc), att.reshape(1, gc), bias.reshape(gc, 1))
    return out_t


def _head_body(ga_ref, go_ref, gm_ref, gv_ref,
               w11, b11, w12, b12, w1p, b1p, g1, bb1,
               w21, b21, w22, b22, g2, bb2,
               w31, b31, w32, b32, w3p, b3p, out_ref):
    def mm(a, w):
        return jnp.dot(a, w[...], preferred_element_type=jnp.float32)

    def bn_tanh(h, g, b):
        mu = jnp.mean(h, axis=0, keepdims=True)
        var = jnp.mean((h - mu) ** 2, axis=0, keepdims=True)
        return jnp.tanh((h - mu) / jnp.sqrt(var + 1e-5) * g[...] + b[...])

    h0 = jnp.concatenate(
        [ga_ref[...], go_ref[...], gm_ref[...], gv_ref[...]], axis=1)
    h = mm(jnp.tanh(mm(h0, w11) + b11[...]), w12) + b12[...]
    h = h + mm(h0, w1p) + b1p[...]
    h = bn_tanh(h, g1, bb1)
    h = h + mm(jnp.tanh(mm(h, w21) + b21[...]), w22) + b22[...]
    h = bn_tanh(h, g2, bb2)
    h3 = mm(jnp.tanh(mm(h, w31) + b31[...]), w32) + b32[...]
    out_ref[...] = h3 + mm(h, w3p) + b3p[...]


def kernel(x_operation, x_machine, x_AGV, global_attr,
           batch_operation, batch_machine, batch_AGV, params):
    p = params
    b = global_attr.shape[0]
    globs = {}
    for t, x, seg in (("operation", x_operation, batch_operation),
                      ("machine", x_machine, batch_machine),
                      ("AGV", x_AGV, batch_AGV)):
        out_t = _gat(x, seg, p["tok_" + t], p["Wl_" + t], p["bl_" + t],
                     p["Wr_" + t], p["br_" + t], p["att_" + t],
                     p["bias_" + t], b)
        globs[t] = out_t.T

    def r1(name):
        return p[name].reshape(1, -1)

    ggc = p["rl3_W1"].shape[1]
    gf = pl.pallas_call(
        _head_body,
        out_shape=jax.ShapeDtypeStruct((b, ggc), jnp.float32),
    )(global_attr, globs["operation"], globs["machine"], globs["AGV"],
      p["rl1_W1"], r1("rl1_b1"), p["rl1_W2"], r1("rl1_b2"),
      p["rl1_Wp"], r1("rl1_bp"), r1("bn1_g"), r1("bn1_b"),
      p["rl2_W1"], r1("rl2_b1"), p["rl2_W2"], r1("rl2_b2"),
      r1("bn2_g"), r1("bn2_b"),
      p["rl3_W1"], r1("rl3_b1"), p["rl3_W2"], r1("rl3_b2"),
      p["rl3_Wp"], r1("rl3_bp"))
    return (globs["operation"], globs["machine"], globs["AGV"], gf)


# speedup vs baseline: 1.3581x; 1.0127x over previous
import functools

import jax
import jax.numpy as jnp
from jax import lax
from jax.experimental import pallas as pl
from jax.experimental.pallas import tpu as pltpu

_NEG = -1e30


def _gat_body(nb, x_ref, seg_ref, wl_ref, bl_ref, tok_ref, wr_ref, br_ref,
              att_ref, bias_ref, out_ref, num_ref, den_ref, m_ref):
    i = pl.program_id(0)

    @pl.when(i == 0)
    def _init():
        num_ref[...] = jnp.zeros(num_ref.shape, jnp.float32)
        den_ref[...] = jnp.zeros(den_ref.shape, jnp.float32)
        m_ref[0, 0] = _NEG

    nseg = den_ref.shape[1]
    r = x_ref.shape[0]
    seg = seg_ref[...]
    x = x_ref[...]
    xl = jnp.dot(x, wl_ref[...], preferred_element_type=jnp.float32) + bl_ref[...]
    valid = seg < nseg
    xl_z = jnp.where(valid, xl, 0.0)
    xr = jnp.dot(tok_ref[...], wr_ref[...], preferred_element_type=jnp.float32) + br_ref[...]
    ms = xl + xr
    s = jnp.where(ms > 0, ms, 0.2 * ms)
    logit = jnp.sum(s * att_ref[...], axis=1, keepdims=True)
    logit = jnp.where(valid, logit, _NEG)

    c = jnp.max(logit)
    m_old = m_ref[0, 0]
    m_new = jnp.maximum(m_old, c)

    @pl.when(c > m_old)
    def _rescale():
        sc = jnp.exp(m_old - m_new)
        den_ref[...] = den_ref[...] * sc
        num_ref[...] = num_ref[...] * sc

    e = jnp.exp(logit - m_new)
    onehot = seg == lax.broadcasted_iota(jnp.int32, (r, nseg), 1)
    me = jnp.where(onehot, e, 0.0)
    ones_r = jnp.ones((1, r), jnp.float32)
    den_ref[...] = den_ref[...] + jnp.dot(
        ones_r, me, preferred_element_type=jnp.float32)
    num_ref[...] = num_ref[...] + lax.dot_general(
        xl_z, me, (((0,), (0,)), ((), ())), preferred_element_type=jnp.float32)
    m_ref[0, 0] = m_new

    @pl.when(i == nb - 1)
    def _finish():
        out_ref[...] = num_ref[...] / den_ref[...] + bias_ref[...]


def _gat(x, seg, tok, wl, bl, wr, br, att, bias, nseg, block_rows=1024):
    n, nc = x.shape
    gc = wl.shape[1]
    nb = -(-n // block_rows)
    pad = nb * block_rows - n
    segp = jnp.concatenate(
        [seg.astype(jnp.int32), jnp.full((pad,), nseg, jnp.int32)]).reshape(-1, 1)
    out_t = pl.pallas_call(
        functools.partial(_gat_body, nb),
        grid=(nb,),
        in_specs=[
            pl.BlockSpec((block_rows, nc), lambda i: (i, 0)),
            pl.BlockSpec((block_rows, 1), lambda i: (i, 0)),
            pl.BlockSpec((nc, gc), lambda i: (0, 0)),
            pl.BlockSpec((1, gc), lambda i: (0, 0)),
            pl.BlockSpec((1, gc), lambda i: (0, 0)),
            pl.BlockSpec((gc, gc), lambda i: (0, 0)),
            pl.BlockSpec((1, gc), lambda i: (0, 0)),
            pl.BlockSpec((1, gc), lambda i: (0, 0)),
            pl.BlockSpec((gc, 1), lambda i: (0, 0)),
        ],
        out_specs=pl.BlockSpec((gc, nseg), lambda i: (0, 0)),
        out_shape=jax.ShapeDtypeStruct((gc, nseg), jnp.float32),
        scratch_shapes=[
            pltpu.VMEM((gc, nseg), jnp.float32),
            pltpu.VMEM((1, nseg), jnp.float32),
            pltpu.SMEM((1, 1), jnp.float32),
        ],
        compiler_params=pltpu.CompilerParams(dimension_semantics=("arbitrary",)),
    )(x, segp, wl, bl.reshape(1, gc), tok.reshape(1, gc), wr,
      br.reshape(1, gc), att.reshape(1, gc), bias.reshape(gc, 1))
    return out_t


def _head_body(ga_ref, go_ref, gm_ref, gv_ref,
               w11, b11, w12, b12, w1p, b1p, g1, bb1,
               w21, b21, w22, b22, g2, bb2,
               w31, b31, w32, b32, w3p, b3p,
               go_out, gm_out, gv_out, gf_out):
    def mm(a, w):
        return jnp.dot(a, w[...], preferred_element_type=jnp.float32)

    def bn_tanh(h, g, b):
        mu = jnp.mean(h, axis=0, keepdims=True)
        var = jnp.mean((h - mu) ** 2, axis=0, keepdims=True)
        return jnp.tanh((h - mu) / jnp.sqrt(var + 1e-5) * g[...] + b[...])

    go = go_ref[...].T
    gm = gm_ref[...].T
    gv = gv_ref[...].T
    go_out[...] = go
    gm_out[...] = gm
    gv_out[...] = gv
    h0 = jnp.concatenate([ga_ref[...], go, gm, gv], axis=1)
    h = mm(jnp.tanh(mm(h0, w11) + b11[...]), w12) + b12[...]
    h = h + mm(h0, w1p) + b1p[...]
    h = bn_tanh(h, g1, bb1)
    h = h + mm(jnp.tanh(mm(h, w21) + b21[...]), w22) + b22[...]
    h = bn_tanh(h, g2, bb2)
    h3 = mm(jnp.tanh(mm(h, w31) + b31[...]), w32) + b32[...]
    gf_out[...] = h3 + mm(h, w3p) + b3p[...]


def kernel(x_operation, x_machine, x_AGV, global_attr,
           batch_operation, batch_machine, batch_AGV, params):
    p = params
    b = global_attr.shape[0]
    globs = {}
    for t, x, seg in (("operation", x_operation, batch_operation),
                      ("machine", x_machine, batch_machine),
                      ("AGV", x_AGV, batch_AGV)):
        globs[t] = _gat(x, seg, p["tok_" + t], p["Wl_" + t], p["bl_" + t],
                        p["Wr_" + t], p["br_" + t], p["att_" + t],
                        p["bias_" + t], b)

    def r1(name):
        return p[name].reshape(1, -1)

    gc = globs["operation"].shape[0]
    ggc = p["rl3_W1"].shape[1]
    go, gm, gv, gf = pl.pallas_call(
        _head_body,
        out_shape=[jax.ShapeDtypeStruct((b, gc), jnp.float32),
                   jax.ShapeDtypeStruct((b, gc), jnp.float32),
                   jax.ShapeDtypeStruct((b, gc), jnp.float32),
                   jax.ShapeDtypeStruct((b, ggc), jnp.float32)],
    )(global_attr, globs["operation"], globs["machine"], globs["AGV"],
      p["rl1_W1"], r1("rl1_b1"), p["rl1_W2"], r1("rl1_b2"),
      p["rl1_Wp"], r1("rl1_bp"), r1("bn1_g"), r1("bn1_b"),
      p["rl2_W1"], r1("rl2_b1"), p["rl2_W2"], r1("rl2_b2"),
      r1("bn2_g"), r1("bn2_b"),
      p["rl3_W1"], r1("rl3_b1"), p["rl3_W2"], r1("rl3_b2"),
      p["rl3_Wp"], r1("rl3_bp"))
    return (go, gm, gv, gf)
